# baseline (device time: 288010 ns/iter reference)
import functools

import jax
import jax.numpy as jnp
from jax import lax
from jax.experimental import pallas as pl
from jax.experimental.pallas import tpu as pltpu

NZ = 4
B, S, H, Dh, Dr = 4, 256, 32, 128, 64
D = 4096
DC = 512
DCZ = DC // NZ
M = B * S
SCALE = (Dh + Dr) ** -0.5

BN = 128
NQ = D // BN
NQR = 2048 // BN
NSTEP = NQ + NQR
WAIT_STEPS = (16, 32, NSTEP - 1)


def _dot(a, b, dn=(((1,), (0,)), ((), ()))):
    return lax.dot_general(
        a, b, dn,
        precision=lax.Precision.DEFAULT,
        preferred_element_type=jnp.float32,
    )


def _gemm_body(x_ref, w_ref, o_ref, *, scale=None):
    r = _dot(x_ref[...], w_ref[...])
    if scale is not None:
        r = r * scale
    o_ref[...] = r.astype(o_ref.dtype)


def _gemm(x, w, block_n=None, scale=None, out_dtype=jnp.float32):
    m, k = x.shape
    _, n = w.shape
    if block_n is None:
        block_n = n
    return pl.pallas_call(
        functools.partial(_gemm_body, scale=scale),
        grid=(n // block_n,),
        in_specs=[
            pl.BlockSpec((m, k), lambda j: (0, 0)),
            pl.BlockSpec((k, block_n), lambda j: (0, j)),
        ],
        out_specs=pl.BlockSpec((m, block_n), lambda j: (0, j)),
        out_shape=jax.ShapeDtypeStruct((m, n), out_dtype),
    )(x, w)


def _cz_body(x_ref, w_ref, cz_ref):
    cz_ref[...] = _dot(x_ref[...], w_ref[...]).astype(jnp.bfloat16)


def _cz(x2, wdkv):
    return pl.pallas_call(
        _cz_body,
        in_specs=[pl.BlockSpec(memory_space=pltpu.VMEM)] * 2,
        out_specs=pl.BlockSpec(memory_space=pltpu.VMEM),
        out_shape=jax.ShapeDtypeStruct((M, DCZ), jnp.bfloat16),
    )(x2, wdkv)


def _gqr_body(x_ref, wq_ref, wqr_ref, czbf_ref, wukz_ref, wuvz_ref,
              qqr_ref, c_all, wukf, wuvf,
              send_c, recv_c, send_k, recv_k, send_v, recv_v):
    j = pl.program_id(0)
    mx = lax.axis_index("x")
    my = lax.axis_index("y")
    mz = lax.axis_index("z")
    right = (mz + 1) % NZ
    left = (mz + NZ - 1) % NZ

    def hop_rdmas(h):
        src = (mz + NZ - h) % NZ
        mk = functools.partial(
            pltpu.make_async_remote_copy,
            device_id=(mx, my, right),
            device_id_type=pl.DeviceIdType.MESH,
        )
        return (
            mk(src_ref=c_all.at[src], dst_ref=c_all.at[src],
               send_sem=send_c.at[h], recv_sem=recv_c.at[h]),
            mk(src_ref=wukf.at[pl.ds(src * DCZ, DCZ)],
               dst_ref=wukf.at[pl.ds(src * DCZ, DCZ)],
               send_sem=send_k.at[h], recv_sem=recv_k.at[h]),
            mk(src_ref=wuvf.at[pl.ds(src * DCZ, DCZ)],
               dst_ref=wuvf.at[pl.ds(src * DCZ, DCZ)],
               send_sem=send_v.at[h], recv_sem=recv_v.at[h]),
        )

    @pl.when(j == 0)
    def _():
        barrier = pltpu.get_barrier_semaphore()
        for nbr in (left, right):
            pl.semaphore_signal(
                barrier, inc=1,
                device_id=(mx, my, nbr),
                device_id_type=pl.DeviceIdType.MESH,
            )
        pl.semaphore_wait(barrier, 2)

        c_all[mz] = czbf_ref[...]
        wukf[pl.ds(mz * DCZ, DCZ), :] = wukz_ref[...].astype(jnp.bfloat16)
        wuvf[pl.ds(mz * DCZ, DCZ), :] = wuvz_ref[...].astype(jnp.bfloat16)
        for r in hop_rdmas(0):
            r.start()

    for h in range(1, NZ):
        @pl.when(j == WAIT_STEPS[h - 1])
        def _(h=h):
            for r in hop_rdmas(h - 1):
                r.wait()
            if h < NZ - 1:
                for r in hop_rdmas(h):
                    r.start()

    @pl.when(j < NQ)
    def _():
        qqr_ref[...] = (
            _dot(x_ref[...], wq_ref[...]) * SCALE
        ).astype(jnp.bfloat16)

    @pl.when(j >= NQ)
    def _():
        qqr_ref[...] = _dot(x_ref[...], wqr_ref[...]).astype(jnp.bfloat16)


def _gather_qqr(x2, wq, wqr, czbf, wuk_z, wuv_z):
    vmem = pl.BlockSpec(memory_space=pltpu.VMEM)
    return pl.pallas_call(
        _gqr_body,
        grid=(NSTEP,),
        in_specs=[
            vmem,
            pl.BlockSpec((D, BN), lambda j: (0, jnp.minimum(j, NQ - 1))),
            pl.BlockSpec((D, BN), lambda j: (0, jnp.maximum(j - NQ, 0))),
            vmem,
            vmem,
            vmem,
        ],
        out_specs=[
            pl.BlockSpec((M, BN), lambda j: (0, j)),
            vmem, vmem, vmem,
        ],
        out_shape=[
            jax.ShapeDtypeStruct((M, D + 2048), jnp.bfloat16),
            jax.ShapeDtypeStruct((NZ, M, DCZ), jnp.bfloat16),
            jax.ShapeDtypeStruct((DC, D), jnp.bfloat16),
            jax.ShapeDtypeStruct((DC, D), jnp.bfloat16),
        ],
        scratch_shapes=[pltpu.SemaphoreType.DMA((NZ - 1,))] * 6,
        compiler_params=pltpu.CompilerParams(
            collective_id=0,
            dimension_semantics=("arbitrary",),
            vmem_limit_bytes=56 * 1024 * 1024,
        ),
    )(x2, wq, wqr, czbf, wuk_z, wuv_z)


def _kv_body(c_ref, w_ref, o_ref):
    acc = _dot(c_ref[0], w_ref[pl.ds(0, DCZ), :])
    for z in range(1, NZ):
        acc = acc + _dot(c_ref[z], w_ref[pl.ds(z * DCZ, DCZ), :])
    o_ref[...] = acc.astype(o_ref.dtype)


def _kv(c_all, w, block_n=1024, out_dtype=jnp.float32):
    return pl.pallas_call(
        _kv_body,
        grid=(D // block_n,),
        in_specs=[
            pl.BlockSpec((NZ, M, DCZ), lambda j: (0, 0, 0)),
            pl.BlockSpec((DC, block_n), lambda j: (0, j)),
        ],
        out_specs=pl.BlockSpec((M, block_n), lambda j: (0, j)),
        out_shape=jax.ShapeDtypeStruct((M, D), out_dtype),
    )(c_all, w)


HPB = 8


def _attn_body(q_ref, qr_ref, k_ref, kr_ref, v_ref, o_ref):
    dn_t = (((1,), (1,)), ((), ()))
    kr = kr_ref[...]
    for i in range(HPB):
        q = q_ref[:, i * Dh:(i + 1) * Dh]
        qr = qr_ref[:, i * Dr:(i + 1) * Dr]
        k = k_ref[:, i * Dh:(i + 1) * Dh]
        v = v_ref[:, i * Dh:(i + 1) * Dh]
        p = jnp.exp(_dot(q, k, dn_t) + _dot(qr, kr, dn_t))
        rs = 1.0 / jnp.sum(p, axis=1, keepdims=True)
        o_ref[:, i * Dh:(i + 1) * Dh] = _dot(p, v) * rs


def _attention(QQr, K, Kr, V):
    qr_off = D // (HPB * Dr)
    return pl.pallas_call(
        _attn_body,
        grid=(B, H // HPB),
        in_specs=[
            pl.BlockSpec((S, HPB * Dh), lambda b, h: (b, h)),
            pl.BlockSpec((S, HPB * Dr), lambda b, h: (b, h + qr_off)),
            pl.BlockSpec((S, HPB * Dh), lambda b, h: (b, h)),
            pl.BlockSpec((S, Dr), lambda b, h: (b, 0)),
            pl.BlockSpec((S, HPB * Dh), lambda b, h: (b, h)),
        ],
        out_specs=pl.BlockSpec((S, HPB * Dh), lambda b, h: (b, h)),
        out_shape=jax.ShapeDtypeStruct((M, D), jnp.float32),
    )(QQr, QQr, K, Kr, V)


def kernel(x, Wdkv, Wuk, Wuv, Wq, Wqr, Wkr, Wo):
    x2 = x.reshape(M, D)
    czbf = _cz(x2, Wdkv)
    QQr, c_all, wuk_f, wuv_f = _gather_qqr(x2, Wq, Wqr, czbf, Wuk, Wuv)
    Kr = _gemm(x2, Wkr, scale=SCALE, out_dtype=jnp.bfloat16)
    K = _kv(c_all, wuk_f, out_dtype=jnp.bfloat16)
    V = _kv(c_all, wuv_f)
    O = _attention(QQr, K, Kr, V)
    out = _gemm(O, Wo, 256)
    return out.reshape(B, S, D)


# device time: 287296 ns/iter; 1.0025x vs baseline; 1.0025x over previous
import functools

import jax
import jax.numpy as jnp
from jax import lax
from jax.experimental import pallas as pl
from jax.experimental.pallas import tpu as pltpu

NZ = 4
B, S, H, Dh, Dr = 4, 256, 32, 128, 64
D = 4096
DC = 512
DCZ = DC // NZ
M = B * S
MH = M // 2
DCZH = DCZ // 2
SCALE = (Dh + Dr) ** -0.5

BN = 128
NQ = D // BN
NQR = 2048 // BN
NSTEP = NQ + NQR
WAIT_STEPS = (16, 32, NSTEP - 1)


def _dot(a, b, dn=(((1,), (0,)), ((), ()))):
    return lax.dot_general(
        a, b, dn,
        precision=lax.Precision.DEFAULT,
        preferred_element_type=jnp.float32,
    )


def _gemm_body(x_ref, w_ref, o_ref, *, scale=None):
    w = w_ref[...]
    if w.dtype != x_ref.dtype:
        w = w.astype(x_ref.dtype)
    r = _dot(x_ref[...], w)
    if scale is not None:
        r = r * scale
    o_ref[...] = r.astype(o_ref.dtype)


def _gemm(x, w, block_n=None, scale=None, out_dtype=jnp.float32):
    m, k = x.shape
    _, n = w.shape
    if block_n is None:
        block_n = n
    return pl.pallas_call(
        functools.partial(_gemm_body, scale=scale),
        grid=(n // block_n,),
        in_specs=[
            pl.BlockSpec((m, k), lambda j: (0, 0)),
            pl.BlockSpec((k, block_n), lambda j: (0, j)),
        ],
        out_specs=pl.BlockSpec((m, block_n), lambda j: (0, j)),
        out_shape=jax.ShapeDtypeStruct((m, n), out_dtype),
    )(x, w)


def _cz_body(x_ref, w_ref, cz_ref, xbf_ref):
    cz_ref[...] = _dot(x_ref[...], w_ref[...]).astype(jnp.bfloat16)
    xbf_ref[...] = (x_ref[...] * SCALE).astype(jnp.bfloat16)


def _cz(x2, wdkv):
    return pl.pallas_call(
        _cz_body,
        in_specs=[pl.BlockSpec(memory_space=pltpu.VMEM)] * 2,
        out_specs=[pl.BlockSpec(memory_space=pltpu.VMEM)] * 2,
        out_shape=[
            jax.ShapeDtypeStruct((M, DCZ), jnp.bfloat16),
            jax.ShapeDtypeStruct((M, D), jnp.bfloat16),
        ],
    )(x2, wdkv)


def _gqr_body(xbf_ref, wq_ref, wqr_ref, czbf_ref, wukz_ref, wuvz_ref,
              qqr_ref, c_all, wukf, wuvf, *sems):
    j = pl.program_id(0)
    mx = lax.axis_index("x")
    my = lax.axis_index("y")
    mz = lax.axis_index("z")
    right = (mz + 1) % NZ
    left = (mz + NZ - 1) % NZ

    def hop_rdmas(h):
        src_cw = (mz + NZ - h) % NZ
        src_cc = (mz + h) % NZ
        mk_r = functools.partial(
            pltpu.make_async_remote_copy,
            device_id=(mx, my, right), device_id_type=pl.DeviceIdType.MESH)
        mk_l = functools.partial(
            pltpu.make_async_remote_copy,
            device_id=(mx, my, left), device_id_type=pl.DeviceIdType.MESH)
        rs = []
        for srcz, mk, s0 in ((src_cw, mk_r, 0), (src_cc, mk_l, 6)):
            rows = pl.ds(0, MH) if s0 == 0 else pl.ds(MH, MH)
            wrows = (pl.ds(srcz * DCZ, DCZH) if s0 == 0
                     else pl.ds(srcz * DCZ + DCZH, DCZH))
            rs.append(mk(src_ref=c_all.at[srcz, rows],
                         dst_ref=c_all.at[srcz, rows],
                         send_sem=sems[s0 + 0].at[h],
                         recv_sem=sems[s0 + 1].at[h]))
            rs.append(mk(src_ref=wukf.at[wrows],
                         dst_ref=wukf.at[wrows],
                         send_sem=sems[s0 + 2].at[h],
                         recv_sem=sems[s0 + 3].at[h]))
            rs.append(mk(src_ref=wuvf.at[wrows],
                         dst_ref=wuvf.at[wrows],
                         send_sem=sems[s0 + 4].at[h],
                         recv_sem=sems[s0 + 5].at[h]))
        return rs

    @pl.when(j == 0)
    def _():
        barrier = pltpu.get_barrier_semaphore()
        for nbr in (left, right):
            pl.semaphore_signal(
                barrier, inc=1,
                device_id=(mx, my, nbr),
                device_id_type=pl.DeviceIdType.MESH,
            )
        pl.semaphore_wait(barrier, 2)

        c_all[mz] = czbf_ref[...]
        wukf[pl.ds(mz * DCZ, DCZ), :] = wukz_ref[...].astype(jnp.bfloat16)
        wuvf[pl.ds(mz * DCZ, DCZ), :] = wuvz_ref[...].astype(jnp.bfloat16)
        for r in hop_rdmas(0):
            r.start()

    for h in range(1, NZ):
        @pl.when(j == WAIT_STEPS[h - 1])
        def _(h=h):
            for r in hop_rdmas(h - 1):
                r.wait()
            if h < NZ - 1:
                for r in hop_rdmas(h):
                    r.start()

    @pl.when(j < NQ)
    def _():
        qqr_ref[...] = _dot(
            xbf_ref[...], wq_ref[...].astype(jnp.bfloat16)
        ).astype(jnp.bfloat16)

    @pl.when(j >= NQ)
    def _():
        qqr_ref[...] = _dot(
            xbf_ref[...], wqr_ref[...].astype(jnp.bfloat16)
        ).astype(jnp.bfloat16)


def _gather_qqr(xbf, wq, wqr, czbf, wuk_z, wuv_z):
    vmem = pl.BlockSpec(memory_space=pltpu.VMEM)
    return pl.pallas_call(
        _gqr_body,
        grid=(NSTEP,),
        in_specs=[
            vmem,
            pl.BlockSpec((D, BN), lambda j: (0, jnp.minimum(j, NQ - 1))),
            pl.BlockSpec((D, BN), lambda j: (0, jnp.maximum(j - NQ, 0))),
            vmem,
            vmem,
            vmem,
        ],
        out_specs=[
            pl.BlockSpec((M, BN), lambda j: (0, j)),
            vmem, vmem, vmem,
        ],
        out_shape=[
            jax.ShapeDtypeStruct((M, D + 2048), jnp.bfloat16),
            jax.ShapeDtypeStruct((NZ, M, DCZ), jnp.bfloat16),
            jax.ShapeDtypeStruct((DC, D), jnp.bfloat16),
            jax.ShapeDtypeStruct((DC, D), jnp.bfloat16),
        ],
        scratch_shapes=[pltpu.SemaphoreType.DMA((NZ - 1,))] * 12,
        compiler_params=pltpu.CompilerParams(
            collective_id=0,
            dimension_semantics=("arbitrary",),
            vmem_limit_bytes=56 * 1024 * 1024,
        ),
    )(xbf, wq, wqr, czbf, wuk_z, wuv_z)


def _kv_body(c_ref, w_ref, o_ref):
    acc = _dot(c_ref[0], w_ref[pl.ds(0, DCZ), :])
    for z in range(1, NZ):
        acc = acc + _dot(c_ref[z], w_ref[pl.ds(z * DCZ, DCZ), :])
    o_ref[...] = acc.astype(o_ref.dtype)


def _kv(c_all, w, block_n=1024, out_dtype=jnp.float32):
    return pl.pallas_call(
        _kv_body,
        grid=(D // block_n,),
        in_specs=[
            pl.BlockSpec((NZ, M, DCZ), lambda j: (0, 0, 0)),
            pl.BlockSpec((DC, block_n), lambda j: (0, j)),
        ],
        out_specs=pl.BlockSpec((M, block_n), lambda j: (0, j)),
        out_shape=jax.ShapeDtypeStruct((M, D), out_dtype),
    )(c_all, w)


HPB = 8


def _attn_body(q_ref, qr_ref, k_ref, kr_ref, v_ref, o_ref):
    dn_t = (((1,), (1,)), ((), ()))
    kr = kr_ref[...]
    for i in range(HPB):
        q = q_ref[:, i * Dh:(i + 1) * Dh]
        qr = qr_ref[:, i * Dr:(i + 1) * Dr]
        k = k_ref[:, i * Dh:(i + 1) * Dh]
        v = v_ref[:, i * Dh:(i + 1) * Dh]
        p = jnp.exp(_dot(q, k, dn_t) + _dot(qr, kr, dn_t))
        rs = 1.0 / jnp.sum(p, axis=1, keepdims=True)
        o_ref[:, i * Dh:(i + 1) * Dh] = (_dot(p, v) * rs).astype(o_ref.dtype)


def _attention(QQr, K, Kr, V):
    qr_off = D // (HPB * Dr)
    return pl.pallas_call(
        _attn_body,
        grid=(B, H // HPB),
        in_specs=[
            pl.BlockSpec((S, HPB * Dh), lambda b, h: (b, h)),
            pl.BlockSpec((S, HPB * Dr), lambda b, h: (b, h + qr_off)),
            pl.BlockSpec((S, HPB * Dh), lambda b, h: (b, h)),
            pl.BlockSpec((S, Dr), lambda b, h: (b, 0)),
            pl.BlockSpec((S, HPB * Dh), lambda b, h: (b, h)),
        ],
        out_specs=pl.BlockSpec((S, HPB * Dh), lambda b, h: (b, h)),
        out_shape=jax.ShapeDtypeStruct((M, D), jnp.bfloat16),
    )(QQr, QQr, K, Kr, V)


def kernel(x, Wdkv, Wuk, Wuv, Wq, Wqr, Wkr, Wo):
    x2 = x.reshape(M, D)
    czbf, xbf = _cz(x2, Wdkv)
    QQr, c_all, wuk_f, wuv_f = _gather_qqr(xbf, Wq, Wqr, czbf, Wuk, Wuv)
    Kr = _gemm(x2, Wkr, out_dtype=jnp.bfloat16)
    K = _kv(c_all, wuk_f, out_dtype=jnp.bfloat16)
    V = _kv(c_all, wuv_f)
    O = _attention(QQr, K, Kr, V)
    out = _gemm(O, Wo, 256)
    return out.reshape(B, S, D)


# device time: 227375 ns/iter; 1.2667x vs baseline; 1.2635x over previous
import functools

import jax
import jax.numpy as jnp
from jax import lax
from jax.experimental import pallas as pl
from jax.experimental.pallas import tpu as pltpu

NZ = 4
B, S, H, Dh, Dr = 4, 256, 32, 128, 64
D = 4096
DC = 512
DCZ = DC // NZ
M = B * S
SCALE = (Dh + Dr) ** -0.5

HPD = H // NZ
NH = HPD * Dh
NHR = HPD * Dr

BN = 128
NQ = NH // BN
NQR = NHR // BN
NSTEP = NQ + NQR

BO = 256
NO = D // BO

_MESH = dict(device_id_type=pl.DeviceIdType.MESH)


def _dot(a, b, dn=(((1,), (0,)), ((), ()))):
    return lax.dot_general(
        a, b, dn,
        precision=lax.Precision.DEFAULT,
        preferred_element_type=jnp.float32,
    )


def _gemm_body(x_ref, w_ref, o_ref, *, scale=None):
    w = w_ref[...]
    if w.dtype != x_ref.dtype:
        w = w.astype(x_ref.dtype)
    r = _dot(x_ref[...], w)
    if scale is not None:
        r = r * scale
    o_ref[...] = r.astype(o_ref.dtype)


def _gemm(x, w, block_n=None, scale=None, out_dtype=jnp.float32):
    m, k = x.shape
    _, n = w.shape
    if block_n is None:
        block_n = n
    return pl.pallas_call(
        functools.partial(_gemm_body, scale=scale),
        grid=(n // block_n,),
        in_specs=[
            pl.BlockSpec((m, k), lambda j: (0, 0)),
            pl.BlockSpec((k, block_n), lambda j: (0, j)),
        ],
        out_specs=pl.BlockSpec((m, block_n), lambda j: (0, j)),
        out_shape=jax.ShapeDtypeStruct((m, n), out_dtype),
    )(x, w)


def _cz_body(x_ref, w_ref, cz_ref, xbf_ref):
    cz_ref[...] = _dot(x_ref[...], w_ref[...]).astype(jnp.bfloat16)
    xbf_ref[...] = (x_ref[...] * SCALE).astype(jnp.bfloat16)


def _cz(x2, wdkv):
    return pl.pallas_call(
        _cz_body,
        in_specs=[pl.BlockSpec(memory_space=pltpu.VMEM)] * 2,
        out_specs=[pl.BlockSpec(memory_space=pltpu.VMEM)] * 2,
        out_shape=[
            jax.ShapeDtypeStruct((M, DCZ), jnp.bfloat16),
            jax.ShapeDtypeStruct((M, D), jnp.bfloat16),
        ],
    )(x2, wdkv)


def _gqr_body(sp_ref, xbf_ref, wq_ref, wqr_ref, czbf_ref, wukz_ref,
              wuvz_ref, qqr_ref, c_all, wuk_own, wuv_own,
              wukbf, wuvbf, send_c, recv_c, send_k, recv_k,
              send_v, recv_v):
    j = pl.program_id(0)
    mx = lax.axis_index("x")
    my = lax.axis_index("y")
    mz = lax.axis_index("z")

    def descriptors():
        ds_ = []
        for r in (1, 2, 3):
            t = (mz + r) % NZ
            dev = dict(device_id=(mx, my, t), **_MESH)
            ds_.append((
                pltpu.make_async_remote_copy(
                    src_ref=czbf_ref, dst_ref=c_all.at[mz],
                    send_sem=send_c.at[r - 1], recv_sem=recv_c.at[r - 1],
                    **dev),
                pltpu.make_async_remote_copy(
                    src_ref=wukbf.at[:, pl.ds(t * NH, NH)],
                    dst_ref=wuk_own.at[pl.ds(mz * DCZ, DCZ)],
                    send_sem=send_k.at[r - 1], recv_sem=recv_k.at[r - 1],
                    **dev),
                pltpu.make_async_remote_copy(
                    src_ref=wuvbf.at[:, pl.ds(t * NH, NH)],
                    dst_ref=wuv_own.at[pl.ds(mz * DCZ, DCZ)],
                    send_sem=send_v.at[r - 1], recv_sem=recv_v.at[r - 1],
                    **dev),
            ))
        return ds_

    @pl.when(j == 0)
    def _():
        barrier = pltpu.get_barrier_semaphore()
        for r in (1, 2, 3):
            pl.semaphore_signal(
                barrier, inc=1, device_id=(mx, my, (mz + r) % NZ), **_MESH)
        pl.semaphore_wait(barrier, 3)

        wukbf[...] = wukz_ref[...].astype(jnp.bfloat16)
        wuvbf[...] = wuvz_ref[...].astype(jnp.bfloat16)
        c_all[mz] = czbf_ref[...]
        wuk_own[pl.ds(mz * DCZ, DCZ), :] = wukbf[:, pl.ds(mz * NH, NH)]
        wuv_own[pl.ds(mz * DCZ, DCZ), :] = wuvbf[:, pl.ds(mz * NH, NH)]
        for trio in descriptors():
            for rd in trio:
                rd.start()

    @pl.when(j == NSTEP - 1)
    def _():
        for trio in descriptors():
            for rd in trio:
                rd.wait()

    @pl.when(j < NQ)
    def _():
        qqr_ref[...] = _dot(
            xbf_ref[...], wq_ref[...].astype(jnp.bfloat16)
        ).astype(jnp.bfloat16)

    @pl.when(j >= NQ)
    def _():
        qqr_ref[...] = _dot(
            xbf_ref[...], wqr_ref[...].astype(jnp.bfloat16)
        ).astype(jnp.bfloat16)


def _gather_qqr(xbf, wq, wqr, czbf, wuk_z, wuv_z, mz):
    vmem = pl.BlockSpec(memory_space=pltpu.VMEM)
    sp = jnp.array([mz], dtype=jnp.int32)
    grid_spec = pltpu.PrefetchScalarGridSpec(
        num_scalar_prefetch=1,
        grid=(NSTEP,),
        in_specs=[
            vmem,
            pl.BlockSpec(
                (D, BN),
                lambda j, sp: (0, sp[0] * NQ + jnp.minimum(j, NQ - 1))),
            pl.BlockSpec(
                (D, BN),
                lambda j, sp: (0, sp[0] * NQR + jnp.maximum(j - NQ, 0))),
            vmem,
            vmem,
            vmem,
        ],
        out_specs=[
            pl.BlockSpec((M, BN), lambda j, sp: (0, j)),
            vmem, vmem, vmem,
        ],
        scratch_shapes=(
            [pltpu.VMEM((DCZ, D), jnp.bfloat16)] * 2
            + [pltpu.SemaphoreType.DMA((3,))] * 6
        ),
    )
    return pl.pallas_call(
        _gqr_body,
        grid_spec=grid_spec,
        out_shape=[
            jax.ShapeDtypeStruct((M, NH + NHR), jnp.bfloat16),
            jax.ShapeDtypeStruct((NZ, M, DCZ), jnp.bfloat16),
            jax.ShapeDtypeStruct((DC, NH), jnp.bfloat16),
            jax.ShapeDtypeStruct((DC, NH), jnp.bfloat16),
        ],
        compiler_params=pltpu.CompilerParams(
            collective_id=0,
            dimension_semantics=("arbitrary",),
            vmem_limit_bytes=56 * 1024 * 1024,
        ),
    )(sp, xbf, wq, wqr, czbf, wuk_z, wuv_z)


def _kv_body(c_ref, wuk_ref, wuv_ref, k_ref, v_ref):
    def acc(w_ref):
        a = _dot(c_ref[0], w_ref[pl.ds(0, DCZ), :])
        for z in range(1, NZ):
            a = a + _dot(c_ref[z], w_ref[pl.ds(z * DCZ, DCZ), :])
        return a
    k_ref[...] = acc(wuk_ref).astype(k_ref.dtype)
    v_ref[...] = acc(wuv_ref)


def _kv(c_all, wuk_own, wuv_own):
    return pl.pallas_call(
        _kv_body,
        in_specs=[pl.BlockSpec(memory_space=pltpu.VMEM)] * 3,
        out_specs=[pl.BlockSpec(memory_space=pltpu.VMEM)] * 2,
        out_shape=[
            jax.ShapeDtypeStruct((M, NH), jnp.bfloat16),
            jax.ShapeDtypeStruct((M, NH), jnp.float32),
        ],
        compiler_params=pltpu.CompilerParams(
            vmem_limit_bytes=56 * 1024 * 1024),
    )(c_all, wuk_own, wuv_own)


def _attn_body(q_ref, qr_ref, k_ref, kr_ref, v_ref, o_ref):
    dn_t = (((1,), (1,)), ((), ()))
    kr = kr_ref[...]
    for i in range(HPD):
        q = q_ref[:, i * Dh:(i + 1) * Dh]
        qr = qr_ref[:, i * Dr:(i + 1) * Dr]
        k = k_ref[:, i * Dh:(i + 1) * Dh]
        v = v_ref[:, i * Dh:(i + 1) * Dh]
        p = jnp.exp(_dot(q, k, dn_t) + _dot(qr, kr, dn_t))
        rs = 1.0 / jnp.sum(p, axis=1, keepdims=True)
        o_ref[:, i * Dh:(i + 1) * Dh] = (_dot(p, v) * rs).astype(o_ref.dtype)


def _attention(QQr, K, Kr, V):
    return pl.pallas_call(
        _attn_body,
        grid=(B,),
        in_specs=[
            pl.BlockSpec((S, NH), lambda b: (b, 0)),
            pl.BlockSpec((S, NHR), lambda b: (b, NH // NHR)),
            pl.BlockSpec((S, NH), lambda b: (b, 0)),
            pl.BlockSpec((S, Dr), lambda b: (b, 0)),
            pl.BlockSpec((S, NH), lambda b: (b, 0)),
        ],
        out_specs=pl.BlockSpec((S, NH), lambda b: (b, 0)),
        out_shape=jax.ShapeDtypeStruct((M, NH), jnp.bfloat16),
    )(QQr, QQr, K, Kr, V)


def _out_body(sp_ref, o_ref, wo_ref, out_ref, o_all,
              send_o, recv_o):
    p = pl.program_id(0)
    j = pl.program_id(1)
    mx = lax.axis_index("x")
    my = lax.axis_index("y")
    mz = lax.axis_index("z")

    def descriptor(r):
        return pltpu.make_async_remote_copy(
            src_ref=o_ref, dst_ref=o_all.at[r - 1],
            send_sem=send_o.at[r - 1], recv_sem=recv_o.at[r - 1],
            device_id=(mx, my, (mz + r) % NZ), **_MESH)

    @pl.when((p == 0) & (j == 0))
    def _():
        barrier = pltpu.get_barrier_semaphore()
        for r in (1, 2, 3):
            pl.semaphore_signal(
                barrier, inc=1, device_id=(mx, my, (mz + r) % NZ), **_MESH)
        pl.semaphore_wait(barrier, 3)
        o_all[NZ - 1] = o_ref[...]
        for r in (1, 2, 3):
            descriptor(r).start()

    for pp in (1, 2, 3):
        @pl.when((p == pp) & (j == 0))
        def _(pp=pp):
            descriptor(pp).wait_recv()

    @pl.when((p == NZ - 1) & (j == NO - 1))
    def _():
        for r in (1, 2, 3):
            descriptor(r).wait_send()

    slot = (p + NZ - 1) % NZ
    contrib = _dot(o_all[slot], wo_ref[...].astype(jnp.bfloat16))
    cols = pl.ds(j * BO, BO)

    @pl.when(p == 0)
    def _():
        out_ref[:, cols] = contrib

    @pl.when(p != 0)
    def _():
        out_ref[:, cols] = out_ref[:, cols] + contrib


def _out_gemm(o_own, wo, mz):
    vmem = pl.BlockSpec(memory_space=pltpu.VMEM)
    sp = (mz - jnp.arange(NZ, dtype=jnp.int32)) % NZ
    grid_spec = pltpu.PrefetchScalarGridSpec(
        num_scalar_prefetch=1,
        grid=(NZ, NO),
        in_specs=[
            vmem,
            pl.BlockSpec((NH, BO), lambda p, j, sp: (sp[p], j)),
        ],
        out_specs=vmem,
        scratch_shapes=[
            pltpu.VMEM((NZ, M, NH), jnp.bfloat16),
            pltpu.SemaphoreType.DMA((3,)),
            pltpu.SemaphoreType.DMA((3,)),
        ],
    )
    return pl.pallas_call(
        _out_body,
        grid_spec=grid_spec,
        out_shape=jax.ShapeDtypeStruct((M, D), jnp.float32),
        compiler_params=pltpu.CompilerParams(
            collective_id=1,
            dimension_semantics=("arbitrary", "arbitrary"),
            vmem_limit_bytes=56 * 1024 * 1024,
        ),
    )(sp, o_own, wo)


def kernel(x, Wdkv, Wuk, Wuv, Wq, Wqr, Wkr, Wo):
    x2 = x.reshape(M, D)
    mz = lax.axis_index("z")
    czbf, xbf = _cz(x2, Wdkv)
    QQr, c_all, wuk_own, wuv_own = _gather_qqr(
        xbf, Wq, Wqr, czbf, Wuk, Wuv, mz)
    Kr = _gemm(x2, Wkr, out_dtype=jnp.bfloat16)
    K, V = _kv(c_all, wuk_own, wuv_own)
    O = _attention(QQr, K, Kr, V)
    out = _out_gemm(O, Wo, mz)
    return out.reshape(B, S, D)


# device time: 210183 ns/iter; 1.3703x vs baseline; 1.0818x over previous
import functools

import jax
import jax.numpy as jnp
from jax import lax
from jax.experimental import pallas as pl
from jax.experimental.pallas import tpu as pltpu

NZ = 4
B, S, H, Dh, Dr = 4, 256, 32, 128, 64
D = 4096
DC = 512
DCZ = DC // NZ
M = B * S
SCALE = (Dh + Dr) ** -0.5

HPD = H // NZ
NH = HPD * Dh
NHR = HPD * Dr

BN = 128
NQ = NH // BN
NQR = NHR // BN
NSTEP = NQ + NQR

BO = 512
NO = D // BO

_MESH = dict(device_id_type=pl.DeviceIdType.MESH)


def _dot(a, b, dn=(((1,), (0,)), ((), ()))):
    return lax.dot_general(
        a, b, dn,
        precision=lax.Precision.DEFAULT,
        preferred_element_type=jnp.float32,
    )


def _gemm_body(x_ref, w_ref, o_ref, *, scale=None):
    w = w_ref[...]
    if w.dtype != x_ref.dtype:
        w = w.astype(x_ref.dtype)
    r = _dot(x_ref[...], w)
    if scale is not None:
        r = r * scale
    o_ref[...] = r.astype(o_ref.dtype)


def _gemm(x, w, block_n=None, scale=None, out_dtype=jnp.float32):
    m, k = x.shape
    _, n = w.shape
    if block_n is None:
        block_n = n
    return pl.pallas_call(
        functools.partial(_gemm_body, scale=scale),
        grid=(n // block_n,),
        in_specs=[
            pl.BlockSpec((m, k), lambda j: (0, 0)),
            pl.BlockSpec((k, block_n), lambda j: (0, j)),
        ],
        out_specs=pl.BlockSpec((m, block_n), lambda j: (0, j)),
        out_shape=jax.ShapeDtypeStruct((m, n), out_dtype),
    )(x, w)


def _cz_body(x_ref, w_ref, cz_ref, xbf_ref):
    cz_ref[...] = _dot(x_ref[...], w_ref[...]).astype(jnp.bfloat16)
    xbf_ref[...] = (x_ref[...] * SCALE).astype(jnp.bfloat16)


def _cz(x2, wdkv):
    return pl.pallas_call(
        _cz_body,
        in_specs=[pl.BlockSpec(memory_space=pltpu.VMEM)] * 2,
        out_specs=[pl.BlockSpec(memory_space=pltpu.VMEM)] * 2,
        out_shape=[
            jax.ShapeDtypeStruct((M, DCZ), jnp.bfloat16),
            jax.ShapeDtypeStruct((M, D), jnp.bfloat16),
        ],
    )(x2, wdkv)


def _gqr_body(sp_ref, xbf_ref, wq_ref, wqr_ref, czbf_ref, wukz_ref,
              wuvz_ref, qqr_ref, c_all, wuk_own, wuv_own,
              wukbf, wuvbf, send_c, recv_c, send_k, recv_k,
              send_v, recv_v):
    j = pl.program_id(0)
    mx = lax.axis_index("x")
    my = lax.axis_index("y")
    mz = lax.axis_index("z")

    def descriptors():
        ds_ = []
        for r in (1, 2, 3):
            t = (mz + r) % NZ
            dev = dict(device_id=(mx, my, t), **_MESH)
            ds_.append((
                pltpu.make_async_remote_copy(
                    src_ref=czbf_ref, dst_ref=c_all.at[mz],
                    send_sem=send_c.at[r - 1], recv_sem=recv_c.at[r - 1],
                    **dev),
                pltpu.make_async_remote_copy(
                    src_ref=wukbf.at[:, pl.ds(t * NH, NH)],
                    dst_ref=wuk_own.at[pl.ds(mz * DCZ, DCZ)],
                    send_sem=send_k.at[r - 1], recv_sem=recv_k.at[r - 1],
                    **dev),
                pltpu.make_async_remote_copy(
                    src_ref=wuvbf.at[:, pl.ds(t * NH, NH)],
                    dst_ref=wuv_own.at[pl.ds(mz * DCZ, DCZ)],
                    send_sem=send_v.at[r - 1], recv_sem=recv_v.at[r - 1],
                    **dev),
            ))
        return ds_

    @pl.when(j == 0)
    def _():
        barrier = pltpu.get_barrier_semaphore()
        for r in (1, 2, 3):
            pl.semaphore_signal(
                barrier, inc=1, device_id=(mx, my, (mz + r) % NZ), **_MESH)
        pl.semaphore_wait(barrier, 3)

        wukbf[...] = wukz_ref[...].astype(jnp.bfloat16)
        wuvbf[...] = wuvz_ref[...].astype(jnp.bfloat16)
        c_all[mz] = czbf_ref[...]
        wuk_own[pl.ds(mz * DCZ, DCZ), :] = wukbf[:, pl.ds(mz * NH, NH)]
        wuv_own[pl.ds(mz * DCZ, DCZ), :] = wuvbf[:, pl.ds(mz * NH, NH)]
        for trio in descriptors():
            for rd in trio:
                rd.start()

    @pl.when(j == NSTEP - 1)
    def _():
        for trio in descriptors():
            for rd in trio:
                rd.wait()

    @pl.when(j < NQ)
    def _():
        qqr_ref[...] = _dot(
            xbf_ref[...], wq_ref[...].astype(jnp.bfloat16)
        ).astype(jnp.bfloat16)

    @pl.when(j >= NQ)
    def _():
        qqr_ref[...] = _dot(
            xbf_ref[...], wqr_ref[...].astype(jnp.bfloat16)
        ).astype(jnp.bfloat16)


def _gather_qqr(xbf, wq, wqr, czbf, wuk_z, wuv_z, mz):
    vmem = pl.BlockSpec(memory_space=pltpu.VMEM)
    sp = jnp.array([mz], dtype=jnp.int32)
    grid_spec = pltpu.PrefetchScalarGridSpec(
        num_scalar_prefetch=1,
        grid=(NSTEP,),
        in_specs=[
            vmem,
            pl.BlockSpec(
                (D, BN),
                lambda j, sp: (0, sp[0] * NQ + jnp.minimum(j, NQ - 1))),
            pl.BlockSpec(
                (D, BN),
                lambda j, sp: (0, sp[0] * NQR + jnp.maximum(j - NQ, 0))),
            vmem,
            vmem,
            vmem,
        ],
        out_specs=[
            pl.BlockSpec((M, BN), lambda j, sp: (0, j)),
            vmem, vmem, vmem,
        ],
        scratch_shapes=(
            [pltpu.VMEM((DCZ, D), jnp.bfloat16)] * 2
            + [pltpu.SemaphoreType.DMA((3,))] * 6
        ),
    )
    return pl.pallas_call(
        _gqr_body,
        grid_spec=grid_spec,
        out_shape=[
            jax.ShapeDtypeStruct((M, NH + NHR), jnp.bfloat16),
            jax.ShapeDtypeStruct((NZ, M, DCZ), jnp.bfloat16),
            jax.ShapeDtypeStruct((DC, NH), jnp.bfloat16),
            jax.ShapeDtypeStruct((DC, NH), jnp.bfloat16),
        ],
        compiler_params=pltpu.CompilerParams(
            collective_id=0,
            dimension_semantics=("arbitrary",),
            vmem_limit_bytes=56 * 1024 * 1024,
        ),
    )(sp, xbf, wq, wqr, czbf, wuk_z, wuv_z)


def _kv_body(c_ref, wuk_ref, wuv_ref, k_ref, v_ref):
    def acc(w_ref):
        a = _dot(c_ref[0], w_ref[pl.ds(0, DCZ), :])
        for z in range(1, NZ):
            a = a + _dot(c_ref[z], w_ref[pl.ds(z * DCZ, DCZ), :])
        return a
    k_ref[...] = acc(wuk_ref).astype(k_ref.dtype)
    v_ref[...] = acc(wuv_ref)


def _kv(c_all, wuk_own, wuv_own):
    return pl.pallas_call(
        _kv_body,
        in_specs=[pl.BlockSpec(memory_space=pltpu.VMEM)] * 3,
        out_specs=[pl.BlockSpec(memory_space=pltpu.VMEM)] * 2,
        out_shape=[
            jax.ShapeDtypeStruct((M, NH), jnp.bfloat16),
            jax.ShapeDtypeStruct((M, NH), jnp.float32),
        ],
        compiler_params=pltpu.CompilerParams(
            vmem_limit_bytes=56 * 1024 * 1024),
    )(c_all, wuk_own, wuv_own)


def _attn_body(q_ref, qr_ref, k_ref, kr_ref, v_ref, o_ref):
    dn_t = (((1,), (1,)), ((), ()))
    kr = kr_ref[...]
    for i in range(HPD):
        q = q_ref[:, i * Dh:(i + 1) * Dh]
        qr = qr_ref[:, i * Dr:(i + 1) * Dr]
        k = k_ref[:, i * Dh:(i + 1) * Dh]
        v = v_ref[:, i * Dh:(i + 1) * Dh]
        p = jnp.exp(_dot(q, k, dn_t) + _dot(qr, kr, dn_t))
        rs = 1.0 / jnp.sum(p, axis=1, keepdims=True)
        o_ref[:, i * Dh:(i + 1) * Dh] = (_dot(p, v) * rs).astype(o_ref.dtype)


def _attention(QQr, K, Kr, V):
    return pl.pallas_call(
        _attn_body,
        grid=(B,),
        in_specs=[
            pl.BlockSpec((S, NH), lambda b: (b, 0)),
            pl.BlockSpec((S, NHR), lambda b: (b, NH // NHR)),
            pl.BlockSpec((S, NH), lambda b: (b, 0)),
            pl.BlockSpec((S, Dr), lambda b: (b, 0)),
            pl.BlockSpec((S, NH), lambda b: (b, 0)),
        ],
        out_specs=pl.BlockSpec((S, NH), lambda b: (b, 0)),
        out_shape=jax.ShapeDtypeStruct((M, NH), jnp.bfloat16),
    )(QQr, QQr, K, Kr, V)


def _out_body(sp_ref, o_ref, wo_ref, out_ref, o_all,
              send_o, recv_o):
    p = pl.program_id(0)
    j = pl.program_id(1)
    mx = lax.axis_index("x")
    my = lax.axis_index("y")
    mz = lax.axis_index("z")

    def descriptor(r):
        return pltpu.make_async_remote_copy(
            src_ref=o_ref, dst_ref=o_all.at[r - 1],
            send_sem=send_o.at[r - 1], recv_sem=recv_o.at[r - 1],
            device_id=(mx, my, (mz + r) % NZ), **_MESH)

    @pl.when((p == 0) & (j == 0))
    def _():
        barrier = pltpu.get_barrier_semaphore()
        for r in (1, 2, 3):
            pl.semaphore_signal(
                barrier, inc=1, device_id=(mx, my, (mz + r) % NZ), **_MESH)
        pl.semaphore_wait(barrier, 3)
        o_all[NZ - 1] = o_ref[...]
        for r in (1, 2, 3):
            descriptor(r).start()

    for pp in (1, 2, 3):
        @pl.when((p == pp) & (j == 0))
        def _(pp=pp):
            descriptor(pp).wait_recv()

    @pl.when((p == NZ - 1) & (j == NO - 1))
    def _():
        for r in (1, 2, 3):
            descriptor(r).wait_send()

    slot = (p + NZ - 1) % NZ
    contrib = _dot(o_all[slot], wo_ref[...].astype(jnp.bfloat16))
    cols = pl.ds(j * BO, BO)

    @pl.when(p == 0)
    def _():
        out_ref[:, cols] = contrib.astype(out_ref.dtype)

    @pl.when(p != 0)
    def _():
        out_ref[:, cols] = (out_ref[:, cols] + contrib).astype(out_ref.dtype)


def _out_gemm(o_own, wo, mz):
    vmem = pl.BlockSpec(memory_space=pltpu.VMEM)
    sp = (mz - jnp.arange(NZ, dtype=jnp.int32)) % NZ
    grid_spec = pltpu.PrefetchScalarGridSpec(
        num_scalar_prefetch=1,
        grid=(NZ, NO),
        in_specs=[
            vmem,
            pl.BlockSpec((NH, BO), lambda p, j, sp: (sp[p], j)),
        ],
        out_specs=vmem,
        scratch_shapes=[
            pltpu.VMEM((NZ, M, NH), jnp.bfloat16),
            pltpu.SemaphoreType.DMA((3,)),
            pltpu.SemaphoreType.DMA((3,)),
        ],
    )
    return pl.pallas_call(
        _out_body,
        grid_spec=grid_spec,
        out_shape=jax.ShapeDtypeStruct((M, D), jnp.bfloat16),
        compiler_params=pltpu.CompilerParams(
            collective_id=1,
            dimension_semantics=("arbitrary", "arbitrary"),
            vmem_limit_bytes=56 * 1024 * 1024,
        ),
    )(sp, o_own, wo)


def kernel(x, Wdkv, Wuk, Wuv, Wq, Wqr, Wkr, Wo):
    x2 = x.reshape(M, D)
    mz = lax.axis_index("z")
    czbf, xbf = _cz(x2, Wdkv)
    QQr, c_all, wuk_own, wuv_own = _gather_qqr(
        xbf, Wq, Wqr, czbf, Wuk, Wuv, mz)
    Kr = _gemm(x2, Wkr, out_dtype=jnp.bfloat16)
    K, V = _kv(c_all, wuk_own, wuv_own)
    O = _attention(QQr, K, Kr, V)
    out = _out_gemm(O, Wo, mz)
    return out.reshape(B, S, D)


# device time: 207975 ns/iter; 1.3848x vs baseline; 1.0106x over previous
import functools

import jax
import jax.numpy as jnp
from jax import lax
from jax.experimental import pallas as pl
from jax.experimental.pallas import tpu as pltpu

NZ = 4
B, S, H, Dh, Dr = 4, 256, 32, 128, 64
D = 4096
DC = 512
DCZ = DC // NZ
M = B * S
SCALE = (Dh + Dr) ** -0.5

HPD = H // NZ
NH = HPD * Dh
NHR = HPD * Dr

BN = 128
NQ = NH // BN
NQR = NHR // BN
NSTEP = NQ + NQR

BO = 1024
NO = D // BO

_MESH = dict(device_id_type=pl.DeviceIdType.MESH)


def _dot(a, b, dn=(((1,), (0,)), ((), ()))):
    return lax.dot_general(
        a, b, dn,
        precision=lax.Precision.DEFAULT,
        preferred_element_type=jnp.float32,
    )


def _gemm_body(x_ref, w_ref, o_ref, *, scale=None):
    w = w_ref[...]
    if w.dtype != x_ref.dtype:
        w = w.astype(x_ref.dtype)
    r = _dot(x_ref[...], w)
    if scale is not None:
        r = r * scale
    o_ref[...] = r.astype(o_ref.dtype)


def _gemm(x, w, block_n=None, scale=None, out_dtype=jnp.float32):
    m, k = x.shape
    _, n = w.shape
    if block_n is None:
        block_n = n
    return pl.pallas_call(
        functools.partial(_gemm_body, scale=scale),
        grid=(n // block_n,),
        in_specs=[
            pl.BlockSpec((m, k), lambda j: (0, 0)),
            pl.BlockSpec((k, block_n), lambda j: (0, j)),
        ],
        out_specs=pl.BlockSpec((m, block_n), lambda j: (0, j)),
        out_shape=jax.ShapeDtypeStruct((m, n), out_dtype),
    )(x, w)


def _cz_body(x_ref, w_ref, cz_ref, xbf_ref):
    cz_ref[...] = _dot(x_ref[...], w_ref[...]).astype(jnp.bfloat16)
    xbf_ref[...] = (x_ref[...] * SCALE).astype(jnp.bfloat16)


def _cz(x2, wdkv):
    return pl.pallas_call(
        _cz_body,
        in_specs=[pl.BlockSpec(memory_space=pltpu.VMEM)] * 2,
        out_specs=[pl.BlockSpec(memory_space=pltpu.VMEM)] * 2,
        out_shape=[
            jax.ShapeDtypeStruct((M, DCZ), jnp.bfloat16),
            jax.ShapeDtypeStruct((M, D), jnp.bfloat16),
        ],
    )(x2, wdkv)


def _gqr_body(sp_ref, xbf_ref, wq_ref, wqr_ref, czbf_ref, wukz_ref,
              wuvz_ref, qqr_ref, c_all, wuk_own, wuv_own,
              wukbf, wuvbf, send_c, recv_c, send_k, recv_k,
              send_v, recv_v):
    j = pl.program_id(0)
    mx = lax.axis_index("x")
    my = lax.axis_index("y")
    mz = lax.axis_index("z")

    def descriptors():
        ds_ = []
        for r in (1, 2, 3):
            t = (mz + r) % NZ
            dev = dict(device_id=(mx, my, t), **_MESH)
            ds_.append((
                pltpu.make_async_remote_copy(
                    src_ref=czbf_ref, dst_ref=c_all.at[mz],
                    send_sem=send_c.at[r - 1], recv_sem=recv_c.at[r - 1],
                    **dev),
                pltpu.make_async_remote_copy(
                    src_ref=wukbf.at[:, pl.ds(t * NH, NH)],
                    dst_ref=wuk_own.at[pl.ds(mz * DCZ, DCZ)],
                    send_sem=send_k.at[r - 1], recv_sem=recv_k.at[r - 1],
                    **dev),
                pltpu.make_async_remote_copy(
                    src_ref=wuvbf.at[:, pl.ds(t * NH, NH)],
                    dst_ref=wuv_own.at[pl.ds(mz * DCZ, DCZ)],
                    send_sem=send_v.at[r - 1], recv_sem=recv_v.at[r - 1],
                    **dev),
            ))
        return ds_

    @pl.when(j == 0)
    def _():
        barrier = pltpu.get_barrier_semaphore()
        for r in (1, 2, 3):
            pl.semaphore_signal(
                barrier, inc=1, device_id=(mx, my, (mz + r) % NZ), **_MESH)
        pl.semaphore_wait(barrier, 3)

        wukbf[...] = wukz_ref[...].astype(jnp.bfloat16)
        wuvbf[...] = wuvz_ref[...].astype(jnp.bfloat16)
        c_all[mz] = czbf_ref[...]
        wuk_own[pl.ds(mz * DCZ, DCZ), :] = wukbf[:, pl.ds(mz * NH, NH)]
        wuv_own[pl.ds(mz * DCZ, DCZ), :] = wuvbf[:, pl.ds(mz * NH, NH)]
        for trio in descriptors():
            for rd in trio:
                rd.start()

    @pl.when(j == NSTEP - 1)
    def _():
        for trio in descriptors():
            for rd in trio:
                rd.wait()

    @pl.when(j < NQ)
    def _():
        qqr_ref[...] = _dot(
            xbf_ref[...], wq_ref[...].astype(jnp.bfloat16)
        ).astype(jnp.bfloat16)

    @pl.when(j >= NQ)
    def _():
        qqr_ref[...] = _dot(
            xbf_ref[...], wqr_ref[...].astype(jnp.bfloat16)
        ).astype(jnp.bfloat16)


def _gather_qqr(xbf, wq, wqr, czbf, wuk_z, wuv_z, mz):
    vmem = pl.BlockSpec(memory_space=pltpu.VMEM)
    sp = jnp.array([mz], dtype=jnp.int32)
    grid_spec = pltpu.PrefetchScalarGridSpec(
        num_scalar_prefetch=1,
        grid=(NSTEP,),
        in_specs=[
            vmem,
            pl.BlockSpec(
                (D, BN),
                lambda j, sp: (0, sp[0] * NQ + jnp.minimum(j, NQ - 1))),
            pl.BlockSpec(
                (D, BN),
                lambda j, sp: (0, sp[0] * NQR + jnp.maximum(j - NQ, 0))),
            vmem,
            vmem,
            vmem,
        ],
        out_specs=[
            pl.BlockSpec((M, BN), lambda j, sp: (0, j)),
            vmem, vmem, vmem,
        ],
        scratch_shapes=(
            [pltpu.VMEM((DCZ, D), jnp.bfloat16)] * 2
            + [pltpu.SemaphoreType.DMA((3,))] * 6
        ),
    )
    return pl.pallas_call(
        _gqr_body,
        grid_spec=grid_spec,
        out_shape=[
            jax.ShapeDtypeStruct((M, NH + NHR), jnp.bfloat16),
            jax.ShapeDtypeStruct((NZ, M, DCZ), jnp.bfloat16),
            jax.ShapeDtypeStruct((DC, NH), jnp.bfloat16),
            jax.ShapeDtypeStruct((DC, NH), jnp.bfloat16),
        ],
        compiler_params=pltpu.CompilerParams(
            collective_id=0,
            dimension_semantics=("arbitrary",),
            vmem_limit_bytes=56 * 1024 * 1024,
        ),
    )(sp, xbf, wq, wqr, czbf, wuk_z, wuv_z)


def _kv_body(c_ref, wuk_ref, wuv_ref, k_ref, v_ref):
    def acc(w_ref):
        a = _dot(c_ref[0], w_ref[pl.ds(0, DCZ), :])
        for z in range(1, NZ):
            a = a + _dot(c_ref[z], w_ref[pl.ds(z * DCZ, DCZ), :])
        return a
    k_ref[...] = acc(wuk_ref).astype(k_ref.dtype)
    v_ref[...] = acc(wuv_ref)


def _kv(c_all, wuk_own, wuv_own):
    return pl.pallas_call(
        _kv_body,
        in_specs=[pl.BlockSpec(memory_space=pltpu.VMEM)] * 3,
        out_specs=[pl.BlockSpec(memory_space=pltpu.VMEM)] * 2,
        out_shape=[
            jax.ShapeDtypeStruct((M, NH), jnp.bfloat16),
            jax.ShapeDtypeStruct((M, NH), jnp.float32),
        ],
        compiler_params=pltpu.CompilerParams(
            vmem_limit_bytes=56 * 1024 * 1024),
    )(c_all, wuk_own, wuv_own)


def _attn_body(q_ref, qr_ref, k_ref, kr_ref, v_ref, o_ref):
    dn_t = (((1,), (1,)), ((), ()))
    kr = kr_ref[...]
    for i in range(HPD):
        q = q_ref[:, i * Dh:(i + 1) * Dh]
        qr = qr_ref[:, i * Dr:(i + 1) * Dr]
        k = k_ref[:, i * Dh:(i + 1) * Dh]
        v = v_ref[:, i * Dh:(i + 1) * Dh]
        p = jnp.exp(_dot(q, k, dn_t) + _dot(qr, kr, dn_t))
        rs = 1.0 / jnp.sum(p, axis=1, keepdims=True)
        o_ref[:, i * Dh:(i + 1) * Dh] = (_dot(p, v) * rs).astype(o_ref.dtype)


def _attention(QQr, K, Kr, V):
    return pl.pallas_call(
        _attn_body,
        grid=(B,),
        in_specs=[
            pl.BlockSpec((S, NH), lambda b: (b, 0)),
            pl.BlockSpec((S, NHR), lambda b: (b, NH // NHR)),
            pl.BlockSpec((S, NH), lambda b: (b, 0)),
            pl.BlockSpec((S, Dr), lambda b: (b, 0)),
            pl.BlockSpec((S, NH), lambda b: (b, 0)),
        ],
        out_specs=pl.BlockSpec((S, NH), lambda b: (b, 0)),
        out_shape=jax.ShapeDtypeStruct((M, NH), jnp.bfloat16),
    )(QQr, QQr, K, Kr, V)


def _out_body(sp_ref, o_ref, wo_ref, out_ref, o_all,
              send_o, recv_o):
    p = pl.program_id(0)
    j = pl.program_id(1)
    mx = lax.axis_index("x")
    my = lax.axis_index("y")
    mz = lax.axis_index("z")

    def descriptor(r):
        return pltpu.make_async_remote_copy(
            src_ref=o_ref, dst_ref=o_all.at[r - 1],
            send_sem=send_o.at[r - 1], recv_sem=recv_o.at[r - 1],
            device_id=(mx, my, (mz + r) % NZ), **_MESH)

    @pl.when((p == 0) & (j == 0))
    def _():
        barrier = pltpu.get_barrier_semaphore()
        for r in (1, 2, 3):
            pl.semaphore_signal(
                barrier, inc=1, device_id=(mx, my, (mz + r) % NZ), **_MESH)
        pl.semaphore_wait(barrier, 3)
        o_all[NZ - 1] = o_ref[...]
        for r in (1, 2, 3):
            descriptor(r).start()

    for pp in (1, 2, 3):
        @pl.when((p == pp) & (j == 0))
        def _(pp=pp):
            descriptor(pp).wait_recv()

    @pl.when((p == NZ - 1) & (j == NO - 1))
    def _():
        for r in (1, 2, 3):
            descriptor(r).wait_send()

    slot = (p + NZ - 1) % NZ
    contrib = _dot(o_all[slot], wo_ref[...].astype(jnp.bfloat16))
    cols = pl.ds(j * BO, BO)

    @pl.when(p == 0)
    def _():
        out_ref[:, cols] = contrib.astype(out_ref.dtype)

    @pl.when(p != 0)
    def _():
        out_ref[:, cols] = (out_ref[:, cols] + contrib).astype(out_ref.dtype)


def _out_gemm(o_own, wo, mz):
    vmem = pl.BlockSpec(memory_space=pltpu.VMEM)
    sp = (mz - jnp.arange(NZ, dtype=jnp.int32)) % NZ
    grid_spec = pltpu.PrefetchScalarGridSpec(
        num_scalar_prefetch=1,
        grid=(NZ, NO),
        in_specs=[
            vmem,
            pl.BlockSpec((NH, BO), lambda p, j, sp: (sp[p], j)),
        ],
        out_specs=vmem,
        scratch_shapes=[
            pltpu.VMEM((NZ, M, NH), jnp.bfloat16),
            pltpu.SemaphoreType.DMA((3,)),
            pltpu.SemaphoreType.DMA((3,)),
        ],
    )
    return pl.pallas_call(
        _out_body,
        grid_spec=grid_spec,
        out_shape=jax.ShapeDtypeStruct((M, D), jnp.bfloat16),
        compiler_params=pltpu.CompilerParams(
            collective_id=1,
            dimension_semantics=("arbitrary", "arbitrary"),
            vmem_limit_bytes=56 * 1024 * 1024,
        ),
    )(sp, o_own, wo)


def kernel(x, Wdkv, Wuk, Wuv, Wq, Wqr, Wkr, Wo):
    x2 = x.reshape(M, D)
    mz = lax.axis_index("z")
    czbf, xbf = _cz(x2, Wdkv)
    QQr, c_all, wuk_own, wuv_own = _gather_qqr(
        xbf, Wq, Wqr, czbf, Wuk, Wuv, mz)
    Kr = _gemm(x2, Wkr, out_dtype=jnp.bfloat16)
    K, V = _kv(c_all, wuk_own, wuv_own)
    O = _attention(QQr, K, Kr, V)
    out = _out_gemm(O, Wo, mz)
    return out.reshape(B, S, D)
